# baseline (speedup 1.0000x reference)
import functools

import jax
import jax.numpy as jnp
from jax.experimental import pallas as pl
from jax.experimental.pallas import tpu as pltpu

_NH = 8


def _dgT(a, b):
    return jax.lax.dot_general(a, b, (((0,), (0,)), ((), ())),
                               preferred_element_type=jnp.float32)


def _dgB(a, b):
    return jax.lax.dot_general(a, b, (((1,), (1,)), ((), ())),
                               preferred_element_type=jnp.float32)


def _dot(a, b):
    return jnp.dot(a, b, preferred_element_type=jnp.float32)


def _bf(x):
    return x.astype(jnp.bfloat16)


def _layernorm(x, g, b, eps):
    mu = jnp.mean(x, axis=-1, keepdims=True)
    var = jnp.mean(jnp.square(x - mu), axis=-1, keepdims=True)
    return (x - mu) * jax.lax.rsqrt(var + eps) * g + b


def _softmax_rows(s):
    m = jnp.max(s, axis=-1, keepdims=True)
    e = jnp.exp(s - m)
    return e / jnp.sum(e, axis=-1, keepdims=True)


_GF = 1


def _front_attn_kernel(x_ref, pos_ref, g_ref, b_ref, t_ref, tc_ref, wq_ref,
                       bq_ref, wk_ref, bk_ref, wv_ref, bv_ref, wks_ref,
                       bks_ref, wvs_ref, bvs_ref, oc_ref, osT_ref, *, eps, hd):
    hd2 = 2 * hd
    row_h = jax.lax.broadcasted_iota(jnp.int32, (hd2, hd2), 0) // hd
    col_h = jax.lax.broadcasted_iota(jnp.int32, (hd2, hd2), 1) // hd
    pair_mask = row_h == col_h
    for g in range(_GF):
        x = x_ref[g]
        xn = _layernorm(x + pos_ref[...], g_ref[...], b_ref[...], eps)

        q = _dot(xn, wq_ref[...]) + bq_ref[...]
        k = _dot(xn, wk_ref[...]) + bk_ref[...]
        v = _dot(_bf(xn), wv_ref[...]) + bv_ref[...]

        ksp = _dgT(k, wks_ref[...]) + bks_ref[...]
        vb = _bf(v)
        vsp = _dgT(vb, wvs_ref[...]) + bvs_ref[...]
        n = x.shape[0]
        for p in range(_NH // 2):
            sl2 = slice(p * hd2, (p + 1) * hd2)
            s = _dgT(q[:, sl2], k[:, sl2]) * tc_ref[sl2, :1]
            s = jnp.where(pair_mask, s, -jnp.inf)
            a = _softmax_rows(s)
            oc2 = _dgB(vb[:, sl2], _bf(a))
            for i in range(2):
                h = 2 * p + i
                sl = slice(h * hd, (h + 1) * hd)
                oc_ref[g, h * n:(h + 1) * n, :] = _bf(
                    oc2[:, i * hd:(i + 1) * hd])
                asp = _softmax_rows(_dot(q[:, sl], ksp[sl, :]) * t_ref[h])
                osT_ref[g, :, h * n:(h + 1) * n] = _bf(
                    _dgB(_bf(vsp[sl, :]), _bf(asp)))


_GT = 8


def _tail_kernel(oc_ref, os_ref, x_ref, pos_ref, g_ref, b_ref,
                 wpa_ref, wpb_ref, bpo_ref, wf1_ref, bf1_ref, wf2_ref,
                 bf2_ref, o_ref, *, eps):
    for g in range(_GT):
        x = x_ref[g]
        xe = x + pos_ref[...]
        xn = _layernorm(xe, g_ref[...], b_ref[...], eps)
        oc = oc_ref[g].astype(jnp.float32) + xn
        osp = os_ref[g].astype(jnp.float32) + xn
        dca = (_dot(_bf(oc), wpa_ref[...]) + _dot(_bf(osp), wpb_ref[...])
               + bpo_ref[...])
        attn = xe + dca
        h1 = jnp.maximum(_dot(_bf(attn), wf1_ref[...]) + bf1_ref[...], 0.0)
        ffn = _dot(_bf(h1), wf2_ref[...]) + bf2_ref[...]
        o_ref[g] = ffn + attn + x


def kernel(x, pos, gamma, beta, temperature, w_qk, b_qk, w_v, b_v,
           w_ks, b_ks, w_vs, b_vs, w_po, b_po, w_f1, b_f1, w_f2, b_f2):
    eps = 1e-5
    B, C, H, W = x.shape
    N = H * W
    hd = C // _NH
    P = w_ks.shape[1]
    bf16 = jnp.bfloat16

    x_tok = jnp.transpose(x.reshape(B, C, N), (0, 2, 1))

    w_q, w_k = w_qk[:, :C], w_qk[:, C:]
    b_q, b_k = b_qk[:, :C], b_qk[:, C:]
    t_chan = jnp.broadcast_to(
        jnp.repeat(temperature.reshape(_NH), hd)[:, None], (C, 128))

    perb = pl.BlockSpec((1, N, C), lambda b: (b, 0, 0))
    full2 = lambda s: pl.BlockSpec(s, lambda b: (0, 0))
    parallel = pltpu.CompilerParams(dimension_semantics=("parallel",))

    oc, osT = pl.pallas_call(
        functools.partial(_front_attn_kernel, eps=eps, hd=hd),
        out_shape=(
            jax.ShapeDtypeStruct((B, _NH * N, hd), bf16),
            jax.ShapeDtypeStruct((B, hd, _NH * N), bf16),
        ),
        grid=(B // _GF,),
        in_specs=[
            pl.BlockSpec((_GF, N, C), lambda b: (b, 0, 0)),
            full2((N, C)),
            full2((1, C)), full2((1, C)),
            pl.BlockSpec(memory_space=pltpu.MemorySpace.SMEM),
            full2((C, 128)),
            full2((C, C)), full2((1, C)),
            full2((C, C)), full2((1, C)),
            full2((C, C)), full2((1, C)),
            full2((N, P)), full2((1, P)),
            full2((N, P)), full2((1, P)),
        ],
        out_specs=(
            pl.BlockSpec((_GF, _NH * N, hd), lambda b: (b, 0, 0)),
            pl.BlockSpec((_GF, hd, _NH * N), lambda b: (b, 0, 0)),
        ),
        compiler_params=parallel,
    )(x_tok, pos, gamma, beta, temperature.reshape(_NH), t_chan, w_q, b_q,
      w_k, b_k, _bf(w_v), b_v, w_ks, b_ks, _bf(w_vs), b_vs)

    out_ch = oc.reshape(B, N, C)
    out_sp = osT.reshape(B, N, C)

    perg = pl.BlockSpec((_GT, N, C), lambda b: (b, 0, 0))
    out_tok = pl.pallas_call(
        functools.partial(_tail_kernel, eps=eps),
        out_shape=jax.ShapeDtypeStruct((B, N, C), jnp.float32),
        grid=(B // _GT,),
        in_specs=[
            perg, perg, perg,
            full2((N, C)),
            full2((1, C)), full2((1, C)),
            full2((C, C)), full2((C, C)), full2((1, C)),
            full2((C, C)), full2((1, C)),
            full2((C, C)), full2((1, C)),
        ],
        out_specs=perg,
        compiler_params=parallel,
    )(out_ch, out_sp, x_tok, pos, gamma, beta, _bf(w_po[:C]), _bf(w_po[C:]),
      b_po, _bf(w_f1), b_f1, _bf(w_f2), b_f2)

    return jnp.transpose(out_tok, (0, 2, 1)).reshape(B, C, H, W)


# speedup vs baseline: 1.0090x; 1.0090x over previous
import functools

import jax
import jax.numpy as jnp
from jax.experimental import pallas as pl
from jax.experimental.pallas import tpu as pltpu

_NH = 8


def _dgT(a, b):
    return jax.lax.dot_general(a, b, (((0,), (0,)), ((), ())),
                               preferred_element_type=jnp.float32)


def _dgB(a, b):
    return jax.lax.dot_general(a, b, (((1,), (1,)), ((), ())),
                               preferred_element_type=jnp.float32)


def _dot(a, b):
    return jnp.dot(a, b, preferred_element_type=jnp.float32)


def _bf(x):
    return x.astype(jnp.bfloat16)


def _layernorm(x, g, b, eps):
    mu = jnp.mean(x, axis=-1, keepdims=True)
    var = jnp.mean(jnp.square(x - mu), axis=-1, keepdims=True)
    return (x - mu) * jax.lax.rsqrt(var + eps) * g + b


def _softmax_rows(s):
    m = jnp.max(s, axis=-1, keepdims=True)
    e = jnp.exp(s - m)
    return e / jnp.sum(e, axis=-1, keepdims=True)


_GF = 1


def _front_attn_kernel(x_ref, pos_ref, g_ref, b_ref, t_ref, tc_ref, wq_ref,
                       bq_ref, wk_ref, bk_ref, wv_ref, bv_ref, wks_ref,
                       bks_ref, wvs_ref, bvs_ref, oc_ref, osT_ref, *, eps, hd):
    hd2 = 2 * hd
    row_h = jax.lax.broadcasted_iota(jnp.int32, (hd2, hd2), 0) // hd
    col_h = jax.lax.broadcasted_iota(jnp.int32, (hd2, hd2), 1) // hd
    pair_mask = row_h == col_h
    for g in range(_GF):
        x = x_ref[g]
        xn = _layernorm(x + pos_ref[...], g_ref[...], b_ref[...], eps)

        q = _dot(xn, wq_ref[...]) + bq_ref[...]
        k = _dot(xn, wk_ref[...]) + bk_ref[...]
        v = _dot(_bf(xn), wv_ref[...]) + bv_ref[...]

        ksp = _dgT(k, wks_ref[...]) + bks_ref[...]
        vb = _bf(v)
        vsp = _dgT(vb, wvs_ref[...]) + bvs_ref[...]
        n = x.shape[0]
        for p in range(_NH // 2):
            sl2 = slice(p * hd2, (p + 1) * hd2)
            s = _dgT(q[:, sl2], k[:, sl2]) * tc_ref[sl2, :1]
            s = jnp.where(pair_mask, s, -jnp.inf)
            a = _softmax_rows(s)
            oc2 = _dgB(vb[:, sl2], _bf(a))
            for i in range(2):
                h = 2 * p + i
                sl = slice(h * hd, (h + 1) * hd)
                oc_ref[g, h * n:(h + 1) * n, :] = _bf(
                    oc2[:, i * hd:(i + 1) * hd])
                asp = _softmax_rows(_dot(q[:, sl], ksp[sl, :]) * t_ref[h])
                osT_ref[g, :, h * n:(h + 1) * n] = _bf(
                    _dgB(_bf(vsp[sl, :]), _bf(asp)))


_GT = 4


def _tail_kernel(oc_ref, os_ref, x_ref, pos_ref, g_ref, b_ref,
                 wpa_ref, wpb_ref, bpo_ref, wf1_ref, bf1_ref, wf2_ref,
                 bf2_ref, o_ref, *, eps):
    for g in range(_GT):
        x = x_ref[g]
        xe = x + pos_ref[...]
        xn = _layernorm(xe, g_ref[...], b_ref[...], eps)
        oc = oc_ref[g].astype(jnp.float32) + xn
        osp = os_ref[g].astype(jnp.float32) + xn
        dca = (_dot(_bf(oc), wpa_ref[...]) + _dot(_bf(osp), wpb_ref[...])
               + bpo_ref[...])
        attn = xe + dca
        h1 = jnp.maximum(_dot(_bf(attn), wf1_ref[...]) + bf1_ref[...], 0.0)
        ffn = _dot(_bf(h1), wf2_ref[...]) + bf2_ref[...]
        o_ref[g] = ffn + attn + x


def kernel(x, pos, gamma, beta, temperature, w_qk, b_qk, w_v, b_v,
           w_ks, b_ks, w_vs, b_vs, w_po, b_po, w_f1, b_f1, w_f2, b_f2):
    eps = 1e-5
    B, C, H, W = x.shape
    N = H * W
    hd = C // _NH
    P = w_ks.shape[1]
    bf16 = jnp.bfloat16

    x_tok = jnp.transpose(x.reshape(B, C, N), (0, 2, 1))

    w_q, w_k = w_qk[:, :C], w_qk[:, C:]
    b_q, b_k = b_qk[:, :C], b_qk[:, C:]
    t_chan = jnp.broadcast_to(
        jnp.repeat(temperature.reshape(_NH), hd)[:, None], (C, 128))

    perb = pl.BlockSpec((1, N, C), lambda b: (b, 0, 0))
    full2 = lambda s: pl.BlockSpec(s, lambda b: (0, 0))
    parallel = pltpu.CompilerParams(dimension_semantics=("parallel",))

    oc, osT = pl.pallas_call(
        functools.partial(_front_attn_kernel, eps=eps, hd=hd),
        out_shape=(
            jax.ShapeDtypeStruct((B, _NH * N, hd), bf16),
            jax.ShapeDtypeStruct((B, hd, _NH * N), bf16),
        ),
        grid=(B // _GF,),
        in_specs=[
            pl.BlockSpec((_GF, N, C), lambda b: (b, 0, 0)),
            full2((N, C)),
            full2((1, C)), full2((1, C)),
            pl.BlockSpec(memory_space=pltpu.MemorySpace.SMEM),
            full2((C, 128)),
            full2((C, C)), full2((1, C)),
            full2((C, C)), full2((1, C)),
            full2((C, C)), full2((1, C)),
            full2((N, P)), full2((1, P)),
            full2((N, P)), full2((1, P)),
        ],
        out_specs=(
            pl.BlockSpec((_GF, _NH * N, hd), lambda b: (b, 0, 0)),
            pl.BlockSpec((_GF, hd, _NH * N), lambda b: (b, 0, 0)),
        ),
        compiler_params=parallel,
    )(x_tok, pos, gamma, beta, temperature.reshape(_NH), t_chan, w_q, b_q,
      w_k, b_k, _bf(w_v), b_v, w_ks, b_ks, _bf(w_vs), b_vs)

    out_ch = oc.reshape(B, N, C)
    out_sp = osT.reshape(B, N, C)

    perg = pl.BlockSpec((_GT, N, C), lambda b: (b, 0, 0))
    out_tok = pl.pallas_call(
        functools.partial(_tail_kernel, eps=eps),
        out_shape=jax.ShapeDtypeStruct((B, N, C), jnp.float32),
        grid=(B // _GT,),
        in_specs=[
            perg, perg, perg,
            full2((N, C)),
            full2((1, C)), full2((1, C)),
            full2((C, C)), full2((C, C)), full2((1, C)),
            full2((C, C)), full2((1, C)),
            full2((C, C)), full2((1, C)),
        ],
        out_specs=perg,
        compiler_params=parallel,
    )(out_ch, out_sp, x_tok, pos, gamma, beta, _bf(w_po[:C]), _bf(w_po[C:]),
      b_po, _bf(w_f1), b_f1, _bf(w_f2), b_f2)

    return jnp.transpose(out_tok, (0, 2, 1)).reshape(B, C, H, W)
